# Initial kernel scaffold; baseline (speedup 1.0000x reference)
import functools

import jax
import jax.numpy as jnp
from jax import lax
from jax.experimental import pallas as pl
from jax.experimental.pallas import tpu as pltpu
from jax.experimental.pallas import tpu_sc as plsc

_NC = 2
_NS = 16
_NW = _NC * _NS
_CH = 80
_F = 64


def _bd4(w):
    return jnp.kron(jnp.eye(4, dtype=w.dtype), w)


def _bt4(b):
    return jnp.tile(b, 4)[None, :]


def _proj(h, w1a, w1b, b1):
    n, dh = h.shape
    hp = h.reshape(n // 4, 4 * dh)

    def body(h_ref, wa_ref, wb_ref, b_ref, pa_ref, pb_ref):
        hv = h_ref[...]
        pa_ref[...] = jnp.dot(hv, wa_ref[...], preferred_element_type=jnp.float32) + b_ref[...]
        pb_ref[...] = jnp.dot(hv, wb_ref[...], preferred_element_type=jnp.float32)

    pa, pb = pl.pallas_call(
        body,
        out_shape=[jax.ShapeDtypeStruct((n // 4, 4 * _F), jnp.float32)] * 2,
    )(hp, _bd4(w1a), _bd4(w1b), _bt4(b1))
    return pa.reshape(n, _F), pb.reshape(n, _F)


def _edge_mlp(ga, gb, e, w1c, lin2, lin3, lin4):
    E = ga.shape[0]
    ep = E // 4
    br = 2000
    grid = ep // br
    row_spec = pl.BlockSpec((br, 4 * _F), lambda i: (i, 0))
    w_spec = pl.BlockSpec((4 * _F, 4 * _F), lambda i: (0, 0))
    b_spec = pl.BlockSpec((1, 4 * _F), lambda i: (0, 0))

    def body(ga_ref, gb_ref, e_ref, w1_ref, w2_ref, b2_ref, w3_ref, b3_ref,
             w4_ref, b4_ref, out_ref):
        z = ga_ref[...] + gb_ref[...] + jnp.dot(
            e_ref[...], w1_ref[...], preferred_element_type=jnp.float32)
        z = jnp.maximum(z, 0.0)
        z = jnp.maximum(jnp.dot(z, w2_ref[...], preferred_element_type=jnp.float32) + b2_ref[...], 0.0)
        z = jnp.maximum(jnp.dot(z, w3_ref[...], preferred_element_type=jnp.float32) + b3_ref[...], 0.0)
        out_ref[...] = jnp.maximum(
            jnp.dot(z, w4_ref[...], preferred_element_type=jnp.float32) + b4_ref[...], 0.0)

    (w2, b2), (w3, b3), (w4, b4) = lin2, lin3, lin4
    out = pl.pallas_call(
        body,
        grid=(grid,),
        in_specs=[row_spec, row_spec, row_spec, w_spec,
                  w_spec, b_spec, w_spec, b_spec, w_spec, b_spec],
        out_specs=row_spec,
        out_shape=jax.ShapeDtypeStruct((ep, 4 * _F), jnp.float32),
    )(ga.reshape(ep, 4 * _F), gb.reshape(ep, 4 * _F), e.reshape(ep, 4 * _F),
      _bd4(w1c), _bd4(w2), _bt4(b2), _bd4(w3), _bt4(b3), _bd4(w4), _bt4(b4))
    return out.reshape(E, _F)


def _node_mlp(agg2, h, v1a, v1b, b1, lin2, lin3, lin4):
    n, dh = h.shape
    nr = n // 4

    def body(a0_ref, a1_ref, h_ref, va_ref, vb_ref, b1_ref,
             w2_ref, b2_ref, w3_ref, b3_ref, w4_ref, b4_ref, out_ref):
        agg = a0_ref[...] + a1_ref[...]
        z = (jnp.dot(agg, va_ref[...], preferred_element_type=jnp.float32)
             + jnp.dot(h_ref[...], vb_ref[...], preferred_element_type=jnp.float32)
             + b1_ref[...])
        z = jnp.maximum(z, 0.0)
        z = jnp.maximum(jnp.dot(z, w2_ref[...], preferred_element_type=jnp.float32) + b2_ref[...], 0.0)
        z = jnp.maximum(jnp.dot(z, w3_ref[...], preferred_element_type=jnp.float32) + b3_ref[...], 0.0)
        out_ref[...] = jnp.maximum(
            jnp.dot(z, w4_ref[...], preferred_element_type=jnp.float32) + b4_ref[...], 0.0)

    (w2, b2), (w3, b3), (w4, b4) = lin2, lin3, lin4
    out = pl.pallas_call(
        body,
        out_shape=jax.ShapeDtypeStruct((nr, 4 * _F), jnp.float32),
    )(agg2[0].reshape(nr, 4 * _F), agg2[1].reshape(nr, 4 * _F),
      h.reshape(nr, 4 * dh), _bd4(v1a), _bd4(v1b), _bt4(b1),
      _bd4(w2), _bt4(b2), _bd4(w3), _bt4(b3), _bd4(w4), _bt4(b4))
    return out.reshape(n, _F)


def _sc_gather(pa, pb, src, dst):
    E = src.shape[0]
    epw = E // _NW
    nch = epw // _CH
    mesh = plsc.VectorSubcoreMesh(core_axis_name="c", subcore_axis_name="s")

    @functools.partial(
        pl.kernel,
        mesh=mesh,
        out_type=[jax.ShapeDtypeStruct((E, _F), jnp.float32)] * 2,
        scratch_types=[
            pltpu.VMEM((_CH,), jnp.int32),
            pltpu.VMEM((_CH, _F), jnp.float32),
            pltpu.VMEM((_CH,), jnp.int32),
            pltpu.VMEM((_CH, _F), jnp.float32),
            pltpu.SemaphoreType.DMA,
            pltpu.SemaphoreType.DMA,
        ],
    )
    def k(pa_h, pb_h, src_h, dst_h, ga_h, gb_h, idxa, bufa, idxb, bufb, sema, semb):
        wid = lax.axis_index("c") * _NS + lax.axis_index("s")
        base = wid * epw

        def chunk(i, carry):
            off = base + i * _CH
            pltpu.sync_copy(src_h.at[pl.ds(off, _CH)], idxa)
            pltpu.sync_copy(dst_h.at[pl.ds(off, _CH)], idxb)
            ca = pltpu.async_copy(pa_h.at[idxa], bufa, sema)
            cb = pltpu.async_copy(pb_h.at[idxb], bufb, semb)
            ca.wait()
            cb.wait()
            pltpu.sync_copy(bufa, ga_h.at[pl.ds(off, _CH)])
            pltpu.sync_copy(bufb, gb_h.at[pl.ds(off, _CH)])
            return carry

        lax.fori_loop(0, nch, chunk, 0)

    return k(pa, pb, src, dst)


def _sc_scatter(ep, dst, zeros):
    E = dst.shape[0]
    n = zeros.shape[0]
    epw = E // _NW
    nch = epw // _CH
    rpt = n // _NS
    mesh = plsc.VectorSubcoreMesh(core_axis_name="c", subcore_axis_name="s")

    @functools.partial(
        pl.kernel,
        mesh=mesh,
        out_type=jax.ShapeDtypeStruct((2 * n, _F), jnp.float32),
        scratch_types=[
            pltpu.VMEM((_CH,), jnp.int32),
            pltpu.VMEM((_CH, _F), jnp.float32),
            pltpu.VMEM((rpt, _F), jnp.float32),
            pltpu.VMEM_SHARED((n, _F), jnp.float32),
        ],
    )
    def k(ep_h, dst_h, z_h, out_h, idx, buf, obuf, acc_sp):
        c = lax.axis_index("c")
        s = lax.axis_index("s")
        base = (c * _NS + s) * epw

        @pl.when(s == 0)
        def _zero():
            pltpu.sync_copy(z_h, acc_sp)

        plsc.subcore_barrier()

        def chunk(i, carry):
            off = base + i * _CH
            pltpu.sync_copy(dst_h.at[pl.ds(off, _CH)], idx)
            pltpu.sync_copy(ep_h.at[pl.ds(off, _CH)], buf)
            pltpu.sync_copy(buf, acc_sp.at[idx], add=True)
            return carry

        lax.fori_loop(0, nch, chunk, 0)
        plsc.subcore_barrier()

        pltpu.sync_copy(acc_sp.at[pl.ds(s * rpt, rpt)], obuf)
        pltpu.sync_copy(obuf, out_h.at[pl.ds(c * n + s * rpt, rpt)])

    out = k(ep, dst, zeros)
    return out.reshape(2, n, _F)


def kernel(x, edge_index, edge_attr, params):
    src = edge_index[0].astype(jnp.int32)
    dst = edge_index[1].astype(jnp.int32)
    n = x.shape[0]
    h = x
    e = edge_attr
    zeros = jnp.zeros((n, _F), jnp.float32)
    for layer in params:
        (w1, b1), l2, l3, l4 = layer["edge"]
        dh = h.shape[1]
        w1a, w1b, w1c = w1[:dh], w1[dh:2 * dh], w1[2 * dh:]
        pa, pb = _proj(h, w1a, w1b, b1)
        ga, gb = _sc_gather(pa, pb, src, dst)
        e = _edge_mlp(ga, gb, e, w1c, l2, l3, l4)
        agg2 = _sc_scatter(e, dst, zeros)
        (v1, nb1), n2, n3, n4 = layer["node"]
        v1a, v1b = v1[:_F], v1[_F:]
        h = _node_mlp(agg2, h, v1a, v1b, nb1, n2, n3, n4)
    return h


# speedup vs baseline: 2.2737x; 2.2737x over previous
import functools

import jax
import jax.numpy as jnp
from jax import lax
from jax.experimental import pallas as pl
from jax.experimental.pallas import tpu as pltpu
from jax.experimental.pallas import tpu_sc as plsc

_NC = 2
_NS = 16
_NW = _NC * _NS
_CH = 80
_F = 64
_NPAD = 10240


def _relu(v):
    return jnp.maximum(v, 0.0)


def _dot(a, b):
    return jnp.dot(a, b, preferred_element_type=jnp.float32)


def _edge_mlp(hs, hd, e, w1a, w1b, w1c, b1, lin2, lin3, lin4):
    E = e.shape[0]
    br = 4000
    grid = E // br
    wide_spec = pl.BlockSpec((br, 2 * _F), lambda i: (i, 0))
    feat_spec = pl.BlockSpec((br, _F), lambda i: (i, 0))

    def cspec(shape):
        return pl.BlockSpec(shape, lambda i: (0, 0))

    def body(hs_ref, hd_ref, e_ref, w1a_ref, w1b_ref, w1c_ref, b1_ref,
             w2_ref, b2_ref, w3_ref, b3_ref, w4_ref, b4_ref, out_ref):
        z = _relu(_dot(hs_ref[...], w1a_ref[...])
                  + _dot(hd_ref[...], w1b_ref[...])
                  + _dot(e_ref[...], w1c_ref[...]) + b1_ref[...])
        z = _relu(_dot(z, w2_ref[...]) + b2_ref[...])
        z = _relu(_dot(z, w3_ref[...]) + b3_ref[...])
        out_ref[...] = _relu(_dot(z, w4_ref[...]) + b4_ref[...])

    (w2, b2), (w3, b3), (w4, b4) = lin2, lin3, lin4
    return pl.pallas_call(
        body,
        grid=(grid,),
        in_specs=[wide_spec, wide_spec, feat_spec,
                  cspec((2 * _F, _F)), cspec((2 * _F, _F)), cspec((_F, _F)),
                  cspec((1, _F)), cspec((_F, _F)), cspec((1, _F)),
                  cspec((_F, _F)), cspec((1, _F)), cspec((_F, _F)),
                  cspec((1, _F))],
        out_specs=feat_spec,
        out_shape=jax.ShapeDtypeStruct((E, _F), jnp.float32),
    )(hs, hd, e, w1a, w1b, w1c, b1[None, :], w2, b2[None, :],
      w3, b3[None, :], w4, b4[None, :])


def _node_mlp(aggf, h, v1a, v1b, b1, lin2, lin3, lin4):
    n, dh = h.shape

    def body(aggf_ref, h_ref, va_ref, vb_ref, b1_ref,
             w2_ref, b2_ref, w3_ref, b3_ref, w4_ref, b4_ref, out_ref):
        agg = aggf_ref[0:n, :] + aggf_ref[_NPAD:_NPAD + n, :]
        z = _relu(_dot(agg, va_ref[...]) + _dot(h_ref[...], vb_ref[...])
                  + b1_ref[...])
        z = _relu(_dot(z, w2_ref[...]) + b2_ref[...])
        z = _relu(_dot(z, w3_ref[...]) + b3_ref[...])
        out_ref[...] = _relu(_dot(z, w4_ref[...]) + b4_ref[...])

    (w2, b2), (w3, b3), (w4, b4) = lin2, lin3, lin4
    return pl.pallas_call(
        body,
        out_shape=jax.ShapeDtypeStruct((n, _F), jnp.float32),
    )(aggf, h, v1a, v1b, b1[None, :], w2, b2[None, :],
      w3, b3[None, :], w4, b4[None, :])


def _sc_gather(hw, src, dst):
    E = src.shape[0]
    epw = E // _NW
    nch = epw // _CH
    mesh = plsc.VectorSubcoreMesh(core_axis_name="c", subcore_axis_name="s")

    @functools.partial(
        pl.kernel,
        mesh=mesh,
        out_type=[jax.ShapeDtypeStruct((E, 2 * _F), jnp.float32)] * 2,
        scratch_types=[
            pltpu.VMEM((_CH,), jnp.int32),
            pltpu.VMEM((_CH, 2 * _F), jnp.float32),
            pltpu.VMEM((_CH,), jnp.int32),
            pltpu.VMEM((_CH, 2 * _F), jnp.float32),
            pltpu.SemaphoreType.DMA,
            pltpu.SemaphoreType.DMA,
        ],
    )
    def k(hw_h, src_h, dst_h, hs_h, hd_h, idxa, bufa, idxb, bufb, sema, semb):
        wid = lax.axis_index("c") * _NS + lax.axis_index("s")
        base = wid * epw

        def chunk(i, carry):
            off = base + i * _CH
            pltpu.sync_copy(src_h.at[pl.ds(off, _CH)], idxa)
            pltpu.sync_copy(dst_h.at[pl.ds(off, _CH)], idxb)
            ca = pltpu.async_copy(hw_h.at[idxa], bufa, sema)
            cb = pltpu.async_copy(hw_h.at[idxb], bufb, semb)
            ca.wait()
            cb.wait()
            pltpu.sync_copy(bufa, hs_h.at[pl.ds(off, _CH)])
            pltpu.sync_copy(bufb, hd_h.at[pl.ds(off, _CH)])
            return carry

        lax.fori_loop(0, nch, chunk, 0)

    return k(hw, src, dst)


def _sc_scatter(ep, dst, zero_ch, rowids):
    E = dst.shape[0]
    epw = E // _NW
    nch = epw // _CH
    rpt = _NPAD // _NS
    mesh = plsc.VectorSubcoreMesh(core_axis_name="c", subcore_axis_name="s")

    @functools.partial(
        pl.kernel,
        mesh=mesh,
        out_type=jax.ShapeDtypeStruct((2 * _NPAD, _F), jnp.float32),
        compiler_params=pltpu.CompilerParams(use_tc_tiling_on_sc=False),
        scratch_types=[
            pltpu.VMEM((_CH,), jnp.int32),
            pltpu.VMEM((_CH, _F), jnp.float32),
            pltpu.VMEM((_CH, _F), jnp.float32),
            pltpu.VMEM_SHARED((_NPAD, _F), jnp.float32),
        ],
    )
    def k(ep_h, dst_h, z_h, id_h, out_h, idx, buf, zbuf, acc):
        c = lax.axis_index("c")
        s = lax.axis_index("s")
        base = (c * _NS + s) * epw

        pltpu.sync_copy(z_h, zbuf)

        def zchunk(j, carry):
            pltpu.sync_copy(id_h.at[pl.ds(s * rpt + j * _CH, _CH)], idx)
            pltpu.sync_copy(zbuf, acc.at[idx])
            return carry

        lax.fori_loop(0, rpt // _CH, zchunk, 0)
        plsc.subcore_barrier()

        def chunk(i, carry):
            off = base + i * _CH
            pltpu.sync_copy(dst_h.at[pl.ds(off, _CH)], idx)
            pltpu.sync_copy(ep_h.at[pl.ds(off, _CH)], buf)
            pltpu.sync_copy(buf, acc.at[idx], add=True)
            return carry

        lax.fori_loop(0, nch, chunk, 0)
        plsc.subcore_barrier()

        pltpu.sync_copy(acc.at[pl.ds(s * rpt, rpt)],
                        out_h.at[pl.ds(c * _NPAD + s * rpt, rpt)])

    return k(ep, dst, zero_ch, rowids)


def kernel(x, edge_index, edge_attr, params):
    src = edge_index[0].astype(jnp.int32)
    dst = edge_index[1].astype(jnp.int32)
    h = x
    e = edge_attr
    zero_ch = jnp.zeros((_CH, _F), jnp.float32)
    rowids = jnp.arange(_NPAD, dtype=jnp.int32)
    for layer in params:
        (w1, b1), l2, l3, l4 = layer["edge"]
        dh = h.shape[1]
        w1a, w1b, w1c = w1[:dh], w1[dh:2 * dh], w1[2 * dh:]
        if dh < 2 * _F:
            hw = jnp.pad(h, ((0, 0), (0, 2 * _F - dh)))
            w1a = jnp.pad(w1a, ((0, 2 * _F - dh), (0, 0)))
            w1b = jnp.pad(w1b, ((0, 2 * _F - dh), (0, 0)))
        else:
            hw = h
        hs, hd = _sc_gather(hw, src, dst)
        e = _edge_mlp(hs, hd, e, w1a, w1b, w1c, b1, l2, l3, l4)
        aggf = _sc_scatter(e, dst, zero_ch, rowids)
        (v1, nb1), n2, n3, n4 = layer["node"]
        v1a, v1b = v1[:_F], v1[_F:]
        h = _node_mlp(aggf, h, v1a, v1b, nb1, n2, n3, n4)
    return h


# speedup vs baseline: 2.9491x; 1.2971x over previous
import functools

import jax
import jax.numpy as jnp
from jax import lax
from jax.experimental import pallas as pl
from jax.experimental.pallas import tpu as pltpu
from jax.experimental.pallas import tpu_sc as plsc

_NC = 2
_NS = 16
_NW = _NC * _NS
_CH = 80
_F = 64
_NPAD = 10240


def _relu(v):
    return jnp.maximum(v, 0.0)


def _dot(a, b):
    return jnp.dot(a, b, preferred_element_type=jnp.float32)


def _edge_mlp(hs, hd, e, w1a, w1b, w1c, b1, lin2, lin3, lin4):
    E = e.shape[0]
    br = 4000
    grid = E // br
    wide_spec = pl.BlockSpec((br, 2 * _F), lambda i: (i, 0))
    feat_spec = pl.BlockSpec((br, _F), lambda i: (i, 0))

    def cspec(shape):
        return pl.BlockSpec(shape, lambda i: (0, 0))

    def body(hs_ref, hd_ref, e_ref, w1a_ref, w1b_ref, w1c_ref, b1_ref,
             w2_ref, b2_ref, w3_ref, b3_ref, w4_ref, b4_ref, out_ref):
        z = _relu(_dot(hs_ref[...], w1a_ref[...])
                  + _dot(hd_ref[...], w1b_ref[...])
                  + _dot(e_ref[...], w1c_ref[...]) + b1_ref[...])
        z = _relu(_dot(z, w2_ref[...]) + b2_ref[...])
        z = _relu(_dot(z, w3_ref[...]) + b3_ref[...])
        out_ref[...] = _relu(_dot(z, w4_ref[...]) + b4_ref[...])

    (w2, b2), (w3, b3), (w4, b4) = lin2, lin3, lin4
    return pl.pallas_call(
        body,
        grid=(grid,),
        in_specs=[wide_spec, wide_spec, feat_spec,
                  cspec((2 * _F, _F)), cspec((2 * _F, _F)), cspec((_F, _F)),
                  cspec((1, _F)), cspec((_F, _F)), cspec((1, _F)),
                  cspec((_F, _F)), cspec((1, _F)), cspec((_F, _F)),
                  cspec((1, _F))],
        out_specs=feat_spec,
        out_shape=jax.ShapeDtypeStruct((E, _F), jnp.float32),
    )(hs, hd, e, w1a, w1b, w1c, b1[None, :],
      w2, b2[None, :], w3, b3[None, :], w4, b4[None, :])


def _node_mlp(aggf, h, v1a, v1b, b1, lin2, lin3, lin4):
    n, dh = h.shape

    def body(aggf_ref, h_ref, va_ref, vb_ref, b1_ref,
             w2_ref, b2_ref, w3_ref, b3_ref, w4_ref, b4_ref, out_ref):
        agg = aggf_ref[0:n, :] + aggf_ref[_NPAD:_NPAD + n, :]
        z = _relu(_dot(agg, va_ref[...]) + _dot(h_ref[...], vb_ref[...])
                  + b1_ref[...])
        z = _relu(_dot(z, w2_ref[...]) + b2_ref[...])
        z = _relu(_dot(z, w3_ref[...]) + b3_ref[...])
        out_ref[...] = _relu(_dot(z, w4_ref[...]) + b4_ref[...])

    (w2, b2), (w3, b3), (w4, b4) = lin2, lin3, lin4
    return pl.pallas_call(
        body,
        out_shape=jax.ShapeDtypeStruct((n, _F), jnp.float32),
    )(aggf, h, v1a, v1b, b1[None, :], w2, b2[None, :],
      w3, b3[None, :], w4, b4[None, :])


def _sc_gather(hw, src, dst):
    E = src.shape[0]
    dt = hw.dtype
    w = hw.shape[1]
    epw = E // _NW
    nch = epw // _CH
    mesh = plsc.VectorSubcoreMesh(core_axis_name="c", subcore_axis_name="s")

    slot_scratch = [
        pltpu.VMEM((_CH,), jnp.int32),
        pltpu.VMEM((_CH,), jnp.int32),
        pltpu.VMEM((_CH, w), dt),
        pltpu.VMEM((_CH, w), dt),
        pltpu.SemaphoreType.DMA,
        pltpu.SemaphoreType.DMA,
    ]

    @functools.partial(
        pl.kernel,
        mesh=mesh,
        out_type=[jax.ShapeDtypeStruct((E, w), dt)] * 2,
        scratch_types=slot_scratch + slot_scratch,
    )
    def k(hw_h, src_h, dst_h, hs_h, hd_h,
          ia0, ib0, ba0, bb0, sa0, sb0, ia1, ib1, ba1, bb1, sa1, sb1):
        wid = lax.axis_index("c") * _NS + lax.axis_index("s")
        base = wid * epw
        slots = ((ia0, ib0, ba0, bb0, sa0, sb0),
                 (ia1, ib1, ba1, bb1, sa1, sb1))

        def prep(i, sl):
            ia, ib, ba, bb, sa, sb = slots[sl]
            off = base + i * _CH
            pltpu.sync_copy(src_h.at[pl.ds(off, _CH)], ia)
            pltpu.sync_copy(dst_h.at[pl.ds(off, _CH)], ib)
            pltpu.async_copy(hw_h.at[ia], ba, sa)
            pltpu.async_copy(hw_h.at[ib], bb, sb)

        def fin(i, sl):
            ia, ib, ba, bb, sa, sb = slots[sl]
            off = base + i * _CH
            pltpu.make_async_copy(hw_h.at[ia], ba, sa).wait()
            pltpu.make_async_copy(hw_h.at[ib], bb, sb).wait()
            pltpu.sync_copy(ba, hs_h.at[pl.ds(off, _CH)])
            pltpu.sync_copy(bb, hd_h.at[pl.ds(off, _CH)])

        prep(0, 0)

        def body(j, carry):
            i0 = 2 * j
            i1 = i0 + 1

            @pl.when(i1 < nch)
            def _p1():
                prep(i1, 1)

            fin(i0, 0)

            @pl.when(i1 + 1 < nch)
            def _p2():
                prep(i1 + 1, 0)

            @pl.when(i1 < nch)
            def _f1():
                fin(i1, 1)

            return carry

        lax.fori_loop(0, (nch + 1) // 2, body, 0)

    return k(hw, src, dst)


def _sc_scatter(ep, dst, zero_ch, rowids):
    E = dst.shape[0]
    epw = E // _NW
    nch = epw // _CH
    rpt = _NPAD // _NS
    mesh = plsc.VectorSubcoreMesh(core_axis_name="c", subcore_axis_name="s")

    @functools.partial(
        pl.kernel,
        mesh=mesh,
        out_type=jax.ShapeDtypeStruct((2 * _NPAD, _F), jnp.float32),
        compiler_params=pltpu.CompilerParams(use_tc_tiling_on_sc=False),
        scratch_types=[
            pltpu.VMEM((_CH,), jnp.int32),
            pltpu.VMEM((_CH, _F), jnp.float32),
            pltpu.SemaphoreType.DMA,
            pltpu.SemaphoreType.DMA,
            pltpu.VMEM((_CH,), jnp.int32),
            pltpu.VMEM((_CH, _F), jnp.float32),
            pltpu.SemaphoreType.DMA,
            pltpu.SemaphoreType.DMA,
            pltpu.VMEM((_CH, _F), jnp.float32),
            pltpu.VMEM_SHARED((_NPAD, _F), jnp.float32),
        ],
    )
    def k(ep_h, dst_h, z_h, id_h, out_h,
          ix0, bf0, sA0, sB0, ix1, bf1, sA1, sB1, zbuf, acc):
        c = lax.axis_index("c")
        s = lax.axis_index("s")
        base = (c * _NS + s) * epw
        slots = ((ix0, bf0, sA0, sB0), (ix1, bf1, sA1, sB1))

        pltpu.sync_copy(z_h, zbuf)

        def zchunk(j, carry):
            pltpu.sync_copy(id_h.at[pl.ds(s * rpt + j * _CH, _CH)], ix0)
            pltpu.sync_copy(zbuf, acc.at[ix0])
            return carry

        lax.fori_loop(0, rpt // _CH, zchunk, 0)
        plsc.subcore_barrier()

        def prep(i, sl):
            idx, buf, sa, sb = slots[sl]
            off = base + i * _CH
            pltpu.async_copy(dst_h.at[pl.ds(off, _CH)], idx, sa)
            pltpu.async_copy(ep_h.at[pl.ds(off, _CH)], buf, sb)

        def fin(i, sl):
            idx, buf, sa, sb = slots[sl]
            off = base + i * _CH
            pltpu.make_async_copy(dst_h.at[pl.ds(off, _CH)], idx, sa).wait()
            pltpu.make_async_copy(ep_h.at[pl.ds(off, _CH)], buf, sb).wait()
            pltpu.sync_copy(buf, acc.at[idx], add=True)

        prep(0, 0)

        def chunk(j, carry):
            i0 = 2 * j
            i1 = i0 + 1

            @pl.when(i1 < nch)
            def _p1():
                prep(i1, 1)

            fin(i0, 0)

            @pl.when(i1 + 1 < nch)
            def _p2():
                prep(i1 + 1, 0)

            @pl.when(i1 < nch)
            def _f1():
                fin(i1, 1)

            return carry

        lax.fori_loop(0, (nch + 1) // 2, chunk, 0)
        plsc.subcore_barrier()

        pltpu.sync_copy(acc.at[pl.ds(s * rpt, rpt)],
                        out_h.at[pl.ds(c * _NPAD + s * rpt, rpt)])

    return k(ep, dst, zero_ch, rowids)


def kernel(x, edge_index, edge_attr, params):
    src = edge_index[0].astype(jnp.int32)
    dst = edge_index[1].astype(jnp.int32)
    h = x
    e = edge_attr
    zero_ch = jnp.zeros((_CH, _F), jnp.float32)
    rowids = jnp.arange(_NPAD, dtype=jnp.int32)
    for layer in params:
        (w1, b1), l2, l3, l4 = layer["edge"]
        dh = h.shape[1]
        w1a, w1b, w1c = w1[:dh], w1[dh:2 * dh], w1[2 * dh:]
        if dh < 2 * _F:
            hw = jnp.pad(h, ((0, 0), (0, 2 * _F - dh)))
            w1a = jnp.pad(w1a, ((0, 2 * _F - dh), (0, 0)))
            w1b = jnp.pad(w1b, ((0, 2 * _F - dh), (0, 0)))
        else:
            hw = h
        hs, hd = _sc_gather(hw, src, dst)
        e = _edge_mlp(hs, hd, e, w1a, w1b, w1c, b1, l2, l3, l4)
        aggf = _sc_scatter(e, dst, zero_ch, rowids)
        (v1, nb1), n2, n3, n4 = layer["node"]
        v1a, v1b = v1[:_F], v1[_F:]
        h = _node_mlp(aggf, h, v1a, v1b, nb1, n2, n3, n4)
    return h
